# Initial kernel scaffold; baseline (speedup 1.0000x reference)
import jax
import jax.numpy as jnp
from jax.experimental import pallas as pl


def kernel(x, edge_index, W1, att_src1, att_dst1, b1, W2, att_src2, att_dst2, b2):
    raise NotImplementedError("write your pallas kernel here")


# speedup vs baseline: 14.7460x; 14.7460x over previous
import functools

import jax
import jax.numpy as jnp
from jax import lax
from jax.experimental import pallas as pl
from jax.experimental.pallas import tpu as pltpu
from jax.experimental.pallas import tpu_sc as plsc

N_NODES = 10000
IN_CH = 128
OUT_CH = 64
HID = 2 * OUT_CH
N_EDGES = 320000

NC = 2
NS = 16
NW = NC * NS
L = 16
CH = 80
ROWS_PW = 128
EPW = ROWS_PW * CH
EPAD = NW * EPW
SB = 32
NPAD = 10240
SEG = NPAD // NS


@functools.partial(
    pl.kernel,
    out_type=(
        jax.ShapeDtypeStruct((NW, ROWS_PW, CH), jnp.float32),
        jax.ShapeDtypeStruct((NW, NPAD), jnp.float32),
    ),
    mesh=plsc.VectorSubcoreMesh(core_axis_name="c", subcore_axis_name="s"),
    compiler_params=pltpu.CompilerParams(needs_layout_passes=False),
    scratch_types=[
        pltpu.VMEM((NPAD,), jnp.float32),
        pltpu.VMEM((NPAD,), jnp.float32),
        pltpu.VMEM((ROWS_PW, CH), jnp.float32),
        pltpu.VMEM((NPAD,), jnp.float32),
        pltpu.VMEM((ROWS_PW, CH), jnp.int32),
        pltpu.VMEM((ROWS_PW, CH), jnp.int32),
    ],
)
def _sc_edge_weights(src_hbm, dst_hbm, a8_hbm, ex_hbm, denp_hbm,
                     asrc, adst, exbuf, denl, sidx, didx):
  c = lax.axis_index("c")
  s = lax.axis_index("s")
  w = c * NS + s
  zv = jnp.zeros((L,), jnp.float32)

  def zden(i, _):
    denl[pl.ds(i * L, L)] = zv
    return 0
  lax.fori_loop(0, NPAD // L, zden, 0)

  pltpu.sync_copy(a8_hbm.at[0], asrc)
  pltpu.sync_copy(a8_hbm.at[1], adst)
  pltpu.sync_copy(src_hbm.at[w], sidx)
  pltpu.sync_copy(dst_hbm.at[w], didx)

  def p1(g, _):
    for j in range(CH // L):
      si = sidx[g, pl.ds(j * L, L)]
      di = didx[g, pl.ds(j * L, L)]
      e = plsc.load_gather(asrc, [si]) + plsc.load_gather(adst, [di])
      e = jnp.where(e >= 0.0, e, 0.2 * e)
      ex = jnp.exp(e)
      exbuf[g, pl.ds(j * L, L)] = ex
      plsc.addupdate_scatter(denl, [di], ex)
    return 0
  lax.fori_loop(0, ROWS_PW, p1, 0)

  pltpu.sync_copy(exbuf, ex_hbm.at[w])
  pltpu.sync_copy(denl, denp_hbm.at[w])


def _sc_gat_aggregate(F):
  @functools.partial(
      pl.kernel,
      out_type=jax.ShapeDtypeStruct((NC, NPAD, F), jnp.float32),
      mesh=plsc.VectorSubcoreMesh(core_axis_name="c", subcore_axis_name="s"),
      compiler_params=pltpu.CompilerParams(needs_layout_passes=False),
      scratch_types=[
          pltpu.VMEM((SB, CH), jnp.int32),
          pltpu.VMEM((SB, CH), jnp.int32),
          pltpu.VMEM((SB, CH), jnp.float32),
          pltpu.VMEM((CH, F), jnp.float32),
          pltpu.VMEM_SHARED((NPAD, F), jnp.float32),
          pltpu.SemaphoreType.DMA,
      ],
  )
  def agg_kernel(src_hbm, dst_hbm, ex_hbm, h_hbm, outp_hbm,
                 sidxb, didxb, exb, rows, sout, sem):
    c = lax.axis_index("c")
    s = lax.axis_index("s")
    w = c * NS + s
    zv = jnp.zeros((L,), jnp.float32)

    def zrow(r, _):
      for q in range(F // L):
        rows[r, pl.ds(q * L, L)] = zv
      return 0
    lax.fori_loop(0, CH, zrow, 0)

    def zout(k, _):
      pltpu.sync_copy(rows, sout.at[pl.ds(s * SEG + k * CH, CH), :])
      return 0
    lax.fori_loop(0, SEG // CH, zout, 0)

    plsc.subcore_barrier()

    def outer(sb, _):
      pltpu.sync_copy(src_hbm.at[w, pl.ds(sb * SB, SB)], sidxb)
      pltpu.sync_copy(dst_hbm.at[w, pl.ds(sb * SB, SB)], didxb)
      pltpu.sync_copy(ex_hbm.at[w, pl.ds(sb * SB, SB)], exb)

      def inner(g, _):
        pltpu.async_copy(h_hbm.at[sidxb.at[g]], rows, sem).wait()

        def scale(r, _):
          av = plsc.load_gather(
              exb, [jnp.full((L,), g, jnp.int32),
                    jnp.full((L,), r, jnp.int32)])
          for q in range(F // L):
            rows[r, pl.ds(q * L, L)] = rows[r, pl.ds(q * L, L)] * av
          return 0
        lax.fori_loop(0, CH, scale, 0)

        pltpu.sync_copy(rows, sout.at[didxb.at[g]], add=True)
        return 0
      lax.fori_loop(0, SB, inner, 0)
      return 0
    lax.fori_loop(0, ROWS_PW // SB, outer, 0)

    plsc.subcore_barrier()

    pltpu.sync_copy(sout.at[pl.ds(s * SEG, SEG), :],
                    outp_hbm.at[c, pl.ds(s * SEG, SEG), :])

  return agg_kernel


_sc_agg = _sc_gat_aggregate(HID)

_NB = 2048


def _embed_body(x_ref, w_ref, att_ref, h_ref, a_ref):
  h = jnp.dot(x_ref[...], w_ref[...], preferred_element_type=jnp.float32)
  h_ref[...] = h
  a_ref[...] = lax.dot_general(att_ref[...], h, (((1,), (1,)), ((), ())),
                               preferred_element_type=jnp.float32)


def _embed(x, W, att8):
  fin = x.shape[1]
  F = W.shape[1]
  return pl.pallas_call(
      _embed_body,
      grid=(NPAD // _NB,),
      in_specs=[
          pl.BlockSpec((_NB, fin), lambda i: (i, 0)),
          pl.BlockSpec((fin, F), lambda i: (0, 0)),
          pl.BlockSpec((8, F), lambda i: (0, 0)),
      ],
      out_specs=[
          pl.BlockSpec((_NB, F), lambda i: (i, 0)),
          pl.BlockSpec((8, _NB), lambda i: (0, i)),
      ],
      out_shape=[
          jax.ShapeDtypeStruct((NPAD, F), jnp.float32),
          jax.ShapeDtypeStruct((8, NPAD), jnp.float32),
      ],
  )(x, W, att8)


def _mid_body(p0_ref, p1_ref, dn_ref, b_ref, w_ref, att_ref, h_ref, a_ref):
  d = jnp.sum(dn_ref[...], axis=0)
  z = p0_ref[...] + p1_ref[...]
  z = z / (d[:, None] + 1e-16) + b_ref[0:1, :]
  z = jnp.maximum(z, 0.0)
  h = jnp.dot(z, w_ref[...], preferred_element_type=jnp.float32)
  h_ref[...] = h
  a_ref[...] = lax.dot_general(att_ref[...], h, (((1,), (1,)), ((), ())),
                               preferred_element_type=jnp.float32)


def _mid(p0, p1, denp, b8, W, att8):
  F = W.shape[1]
  return pl.pallas_call(
      _mid_body,
      grid=(NPAD // _NB,),
      in_specs=[
          pl.BlockSpec((_NB, HID), lambda i: (i, 0)),
          pl.BlockSpec((_NB, HID), lambda i: (i, 0)),
          pl.BlockSpec((NW, _NB), lambda i: (0, i)),
          pl.BlockSpec((8, HID), lambda i: (0, 0)),
          pl.BlockSpec((HID, F), lambda i: (0, 0)),
          pl.BlockSpec((8, F), lambda i: (0, 0)),
      ],
      out_specs=[
          pl.BlockSpec((_NB, F), lambda i: (i, 0)),
          pl.BlockSpec((8, _NB), lambda i: (0, i)),
      ],
      out_shape=[
          jax.ShapeDtypeStruct((NPAD, F), jnp.float32),
          jax.ShapeDtypeStruct((8, NPAD), jnp.float32),
      ],
  )(p0, p1, denp, b8, W, att8)


def _fin_body(p0_ref, p1_ref, dn_ref, b_ref, o_ref):
  d = jnp.sum(dn_ref[...], axis=0)
  z = p0_ref[...] + p1_ref[...]
  o_ref[...] = z / (d[:, None] + 1e-16) + b_ref[0:1, :]


def _fin(p0, p1, denp, b8):
  F = p0.shape[1]
  return pl.pallas_call(
      _fin_body,
      grid=(NPAD // _NB,),
      in_specs=[
          pl.BlockSpec((_NB, F), lambda i: (i, 0)),
          pl.BlockSpec((_NB, F), lambda i: (i, 0)),
          pl.BlockSpec((NW, _NB), lambda i: (0, i)),
          pl.BlockSpec((8, F), lambda i: (0, 0)),
      ],
      out_specs=pl.BlockSpec((_NB, F), lambda i: (i, 0)),
      out_shape=jax.ShapeDtypeStruct((NPAD, F), jnp.float32),
  )(p0, p1, denp, b8)


def kernel(x, edge_index, W1, att_src1, att_dst1, b1,
           W2, att_src2, att_dst2, b2):
  pad = jnp.full((EPAD - N_EDGES,), NPAD - 1, jnp.int32)
  src2d = jnp.concatenate(
      [edge_index[0].astype(jnp.int32), pad]).reshape(NW, ROWS_PW, CH)
  dst2d = jnp.concatenate(
      [edge_index[1].astype(jnp.int32), pad]).reshape(NW, ROWS_PW, CH)

  att8_1 = jnp.concatenate(
      [att_src1[None], att_dst1[None], jnp.zeros((6, HID), jnp.float32)])
  att8_2 = jnp.concatenate(
      [jnp.pad(att_src2[None], ((0, 0), (0, HID - OUT_CH))),
       jnp.pad(att_dst2[None], ((0, 0), (0, HID - OUT_CH))),
       jnp.zeros((6, HID), jnp.float32)])
  b8_1 = jnp.broadcast_to(b1[None], (8, HID))
  b8_2 = jnp.broadcast_to(jnp.pad(b2, (0, HID - OUT_CH))[None], (8, HID))
  W2p = jnp.pad(W2, ((0, 0), (0, HID - OUT_CH)))

  xp = jnp.pad(x, ((0, NPAD - N_NODES), (0, 0)))
  h1, a1 = _embed(xp, W1, att8_1)
  ex1, denp1 = _sc_edge_weights(src2d, dst2d, a1)
  outp1 = _sc_agg(src2d, dst2d, ex1, h1)
  h2, a2 = _mid(outp1[0], outp1[1], denp1, b8_1, W2p, att8_2)
  ex2, denp2 = _sc_edge_weights(src2d, dst2d, a2)
  outp2 = _sc_agg(src2d, dst2d, ex2, h2)
  out = _fin(outp2[0], outp2[1], denp2, b8_2)
  return out[:N_NODES, :OUT_CH]
